# Initial kernel scaffold; baseline (speedup 1.0000x reference)
import jax
import jax.numpy as jnp
from jax.experimental import pallas as pl


def kernel(combine_x, pk_A, pk_b, pk_ln_w, pk_ln_b, trade_A, trade_b, trade_ln_w, trade_ln_b):
    raise NotImplementedError("write your pallas kernel here")


# speedup vs baseline: 4.6001x; 4.6001x over previous
import functools

import jax
import jax.numpy as jnp
from jax import lax
from jax.experimental import pallas as pl
from jax.experimental.pallas import tpu as pltpu

_TB = 8


def _fused_kernel(x_ref, a_ref, b_ref, m_ref, g_ref, be_ref, o_ref, *, eps, tb, h, w):
    x = x_ref[...].reshape(tb * h, w)
    zt = lax.dot_general(a_ref[...], x, (((0,), (1,)), ((), ())),
                         preferred_element_type=jnp.float32)
    zt = zt + b_ref[...]
    m = m_ref[...]
    mean = jnp.dot(m, zt, preferred_element_type=jnp.float32)
    zc = zt - mean
    var = jnp.dot(m, zc * zc, preferred_element_type=jnp.float32)
    y = zc * lax.rsqrt(var + eps) * g_ref[...] + be_ref[...]
    for b in range(tb):
        o_ref[b] = y[:, b * h:(b + 1) * h]


def kernel(combine_x, pk_A, pk_b, pk_ln_w, pk_ln_b,
           trade_A, trade_b, trade_ln_w, trade_ln_b):
    eps = 1e-6
    bsz, _, h, w = combine_x.shape
    kp, cp = pk_A.shape
    kt, ct = trade_A.shape
    c = cp + ct

    a = jnp.zeros((w, c), jnp.float32)
    a = a.at[:kp, :cp].set(pk_A.astype(jnp.float32))
    a = a.at[kp:kp + kt, cp:].set(trade_A.astype(jnp.float32))
    bias = jnp.concatenate([pk_b, trade_b]).astype(jnp.float32).reshape(c, 1)
    gamma = jnp.concatenate([pk_ln_w, trade_ln_w]).astype(jnp.float32).reshape(c, 1)
    beta = jnp.concatenate([pk_ln_b, trade_ln_b]).astype(jnp.float32).reshape(c, 1)
    grp = jnp.arange(c) >= cp
    same = grp[:, None] == grp[None, :]
    inv = jnp.where(grp, 1.0 / ct, 1.0 / cp)
    m = jnp.where(same, inv[None, :], 0.0).astype(jnp.float32)

    x3 = combine_x.reshape(bsz, h, w)
    tb = _TB
    grid = (bsz // tb,)
    out = pl.pallas_call(
        functools.partial(_fused_kernel, eps=eps, tb=tb, h=h, w=w),
        out_shape=jax.ShapeDtypeStruct((bsz, c, h), combine_x.dtype),
        grid_spec=pltpu.PrefetchScalarGridSpec(
            num_scalar_prefetch=0,
            grid=grid,
            in_specs=[
                pl.BlockSpec((tb, h, w), lambda i: (i, 0, 0)),
                pl.BlockSpec((w, c), lambda i: (0, 0)),
                pl.BlockSpec((c, 1), lambda i: (0, 0)),
                pl.BlockSpec((c, c), lambda i: (0, 0)),
                pl.BlockSpec((c, 1), lambda i: (0, 0)),
                pl.BlockSpec((c, 1), lambda i: (0, 0)),
            ],
            out_specs=pl.BlockSpec((tb, c, h), lambda i: (i, 0, 0)),
        ),
        compiler_params=pltpu.CompilerParams(
            dimension_semantics=("parallel",),
            vmem_limit_bytes=64 * 1024 * 1024),
        cost_estimate=pl.CostEstimate(
            flops=int(2 * bsz * h * w * c + 4 * bsz * h * c * c + 10 * bsz * h * c),
            transcendentals=int(bsz * h),
            bytes_accessed=int(4 * bsz * h * (w + c))),
    )(x3, a, bias, m, gamma, beta)
    return out.reshape(bsz, c, h, 1)
